# baseline (speedup 1.0000x reference)
import functools

import jax
import jax.numpy as jnp
from jax import lax
from jax.experimental import pallas as pl
from jax.experimental.pallas import tpu as pltpu
from jax.experimental.pallas import tpu_sc as plsc

N = 50000
E = 800000
D = 100
H = 2
C = 50
TDIM = 100
MSG = 100
W = 128
CW = 112


PB = 2000


def _proj_kernel(x_ref, wq_ref, bq_ref, wk_ref, bk_ref, wv_ref, bv_ref,
                 ws_ref, bs_ref, q_ref, k_ref, v_ref, skip_ref):
    xb = x_ref[:]
    scale = 1.0 / (C ** 0.5)
    q_ref[:] = (jnp.dot(xb, wq_ref[:], preferred_element_type=jnp.float32)
                + bq_ref[:]) * scale
    k_ref[:] = jnp.dot(xb, wk_ref[:], preferred_element_type=jnp.float32) + bk_ref[:]
    v_ref[:] = jnp.dot(xb, wv_ref[:], preferred_element_type=jnp.float32) + bv_ref[:]
    skip_ref[:] = jnp.dot(xb, ws_ref[:], preferred_element_type=jnp.float32) + bs_ref[:]


def _projections(x, Wq, bq, Wk, bk, Wv, bv, Wskip, bskip):
    const = lambda i: (0, 0)
    blk = lambda i: (i, 0)
    w_spec = pl.BlockSpec((D, W), const)
    b_spec = pl.BlockSpec((1, W), const)
    o_spec = pl.BlockSpec((PB, W), blk)
    pad_w = lambda w: jnp.pad(w, ((0, 0), (0, W - D)))
    pad_b = lambda b: jnp.pad(b.reshape(1, D), ((0, 0), (0, W - D)))
    return pl.pallas_call(
        _proj_kernel,
        grid=(N // PB,),
        in_specs=[pl.BlockSpec((PB, D), blk),
                  w_spec, b_spec, w_spec, b_spec, w_spec, b_spec,
                  pl.BlockSpec((D, D), const), pl.BlockSpec((1, D), const)],
        out_specs=[o_spec, o_spec, o_spec, pl.BlockSpec((PB, D), blk)],
        out_shape=[jax.ShapeDtypeStruct((N, W), jnp.float32),
                   jax.ShapeDtypeStruct((N, W), jnp.float32),
                   jax.ShapeDtypeStruct((N, W), jnp.float32),
                   jax.ShapeDtypeStruct((N, D), jnp.float32)],
        compiler_params=pltpu.CompilerParams(
            dimension_semantics=("arbitrary",),
        ),
    )(x, pad_w(Wq), pad_b(bq), pad_w(Wk), pad_b(bk),
      pad_w(Wv), pad_b(bv), Wskip, bskip.reshape(1, D))


NW = 32
GQ = 128
NG = E // GQ
GPW = (NG + NW - 1) // NW


def _gather_body(k_hbm, v_hbm, q_hbm, lu_hbm, src_hbm, dst_hbm,
                 ks_hbm, vs_hbm, qd_hbm, lus_hbm,
                 sidx_v, didx_v, k_v, v_v, q_v, lu_v,
                 sem1, sem2, sem3, sem4):
    c = lax.axis_index("c")
    s = lax.axis_index("s")
    wid = s * 2 + c

    def body(i, _):
        g = wid + NW * i

        @pl.when(g < NG)
        def _():
            base = g * GQ
            pltpu.sync_copy(src_hbm.at[pl.ds(base, GQ)], sidx_v)
            pltpu.sync_copy(dst_hbm.at[pl.ds(base, GQ)], didx_v)
            cp1 = pltpu.async_copy(k_hbm.at[sidx_v], k_v, sem1)
            cp2 = pltpu.async_copy(v_hbm.at[sidx_v], v_v, sem2)
            cp3 = pltpu.async_copy(q_hbm.at[didx_v], q_v, sem3)
            cp4 = pltpu.async_copy(lu_hbm.at[sidx_v], lu_v, sem4)
            cp1.wait()
            cp2.wait()
            cp3.wait()
            cp4.wait()
            pltpu.sync_copy(k_v, ks_hbm.at[pl.ds(base, GQ)])
            pltpu.sync_copy(v_v, vs_hbm.at[pl.ds(base, GQ)])
            pltpu.sync_copy(q_v, qd_hbm.at[pl.ds(base, GQ)])
            pltpu.sync_copy(lu_v, lus_hbm.at[pl.ds(base, GQ)])
        return 0

    lax.fori_loop(0, GPW, body, 0)


def _sc_gather(k, v, q, lu, src, dst):
    mesh = plsc.VectorSubcoreMesh(core_axis_name="c", subcore_axis_name="s")
    fn = pl.kernel(
        _gather_body,
        mesh=mesh,
        out_type=[
            jax.ShapeDtypeStruct((E, W), jnp.float32),
            jax.ShapeDtypeStruct((E, W), jnp.float32),
            jax.ShapeDtypeStruct((E, W), jnp.float32),
            jax.ShapeDtypeStruct((E,), jnp.float32),
        ],
        scratch_types=[
            pltpu.VMEM((GQ,), jnp.int32),
            pltpu.VMEM((GQ,), jnp.int32),
            pltpu.VMEM((GQ, W), jnp.float32),
            pltpu.VMEM((GQ, W), jnp.float32),
            pltpu.VMEM((GQ, W), jnp.float32),
            pltpu.VMEM((GQ,), jnp.float32),
            pltpu.SemaphoreType.DMA,
            pltpu.SemaphoreType.DMA,
            pltpu.SemaphoreType.DMA,
            pltpu.SemaphoreType.DMA,
        ],
    )
    return fn(k, v, q, lu, src, dst)


EB = 2000
NB = E // EB


def _edge_pass_kernel(lus_ref, t_ref, msg_ref, qd_ref, ks_ref, vs_ref,
                      wt_ref, bt_ref, we1_ref, we2_ref, contrib_ref):
    rel = lus_ref[:] - t_ref[:]
    enc = jnp.cos(rel * wt_ref[:] + bt_ref[:])
    e = (jnp.dot(enc, we1_ref[:], preferred_element_type=jnp.float32)
         + jnp.dot(msg_ref[:], we2_ref[:], preferred_element_type=jnp.float32))
    kj = ks_ref[:] + e
    vj = vs_ref[:] + e
    prod = qd_ref[:] * kj
    lane = lax.broadcasted_iota(jnp.int32, (EB, W), 1)
    m0 = lane < C
    a0 = jnp.sum(jnp.where(m0, prod, 0.0), axis=1, keepdims=True)
    a1 = jnp.sum(jnp.where(m0, 0.0, prod), axis=1, keepdims=True)
    ex0 = jnp.exp(a0)
    ex1 = jnp.exp(a1)
    attnw = jnp.where(m0, ex0, ex1)
    base = attnw * vj
    base = jnp.where(lane == D + 1, ex0, base)
    base = jnp.where(lane == D + 2, ex1, base)
    contrib_ref[:] = base[:, :CW]


def _edge_pass(lus, t, msg, qd, ks, vs, wt, bt, we1, we2):
    const = lambda i: (0, 0)
    blk = lambda i: (i, 0)
    return pl.pallas_call(
        _edge_pass_kernel,
        grid=(NB,),
        in_specs=[
            pl.BlockSpec((EB, 1), blk),
            pl.BlockSpec((EB, 1), blk),
            pl.BlockSpec((EB, MSG), blk),
            pl.BlockSpec((EB, W), blk),
            pl.BlockSpec((EB, W), blk),
            pl.BlockSpec((EB, W), blk),
            pl.BlockSpec((1, W), const),
            pl.BlockSpec((1, W), const),
            pl.BlockSpec((W, W), const),
            pl.BlockSpec((MSG, W), const),
        ],
        out_specs=pl.BlockSpec((EB, CW), blk),
        out_shape=jax.ShapeDtypeStruct((E, CW), jnp.float32),
        compiler_params=pltpu.CompilerParams(
            dimension_semantics=("arbitrary",),
        ),
    )(lus, t, msg, qd, ks, vs, wt, bt, we1, we2)


PROWS = 12544
ACC_ROWS = PROWS
ZCHUNK = ACC_ROWS // 16
DCHUNK = 784
OUT_ROWS = 4 * PROWS


def _scatter_body(contrib_hbm, idxp_hbm, zeros_hbm, out_hbm,
                  rows0, idx_v, acc, semr0, semi0):
    c = lax.axis_index("c")
    s = lax.axis_index("s")

    SNI = (NG + 15) // 16

    for p_local in range(2):
        pp = 2 * c + p_local
        pbase = pp * PROWS
        pltpu.sync_copy(zeros_hbm, acc.at[pl.ds(s * ZCHUNK, ZCHUNK)])
        plsc.subcore_barrier()

        def body(i, _):
            g = s + 16 * i

            @pl.when(g < NG)
            def _():
                base = g * GQ
                row = pp * NG + g
                cpi = pltpu.async_copy(idxp_hbm.at[pl.ds(row, 1)],
                                       idx_v, semi0)
                cpr = pltpu.async_copy(contrib_hbm.at[pl.ds(base, GQ)], rows0,
                                       semr0)
                cpi.wait()
                cpr.wait()
                pltpu.sync_copy(rows0,
                                acc.at[plsc.Indices(idx_v.at[0],
                                                    ignored_value=-1)],
                                add=True)
            return 0

        lax.fori_loop(0, SNI, body, 0)
        plsc.subcore_barrier()
        pltpu.sync_copy(
            acc.at[pl.ds(s * DCHUNK, DCHUNK)],
            out_hbm.at[pl.ds(pbase + s * DCHUNK, DCHUNK)])
        plsc.subcore_barrier()


def _sc_copy_body(contrib_hbm, out_hbm, rows0, semr0):
    c = lax.axis_index("c")
    s = lax.axis_index("s")
    wid = s * 2 + c

    def body(i, _):
        g = wid + NW * i

        @pl.when(g < NG)
        def _():
            base = g * GQ
            pltpu.async_copy(contrib_hbm.at[pl.ds(base, GQ)], rows0,
                             semr0).wait()
            pltpu.sync_copy(rows0, out_hbm.at[pl.ds(base, GQ)])
        return 0

    lax.fori_loop(0, GPW, body, 0)


def _sc_copy(contrib):
    mesh = plsc.VectorSubcoreMesh(core_axis_name="c", subcore_axis_name="s")
    fn = pl.kernel(
        _sc_copy_body,
        mesh=mesh,
        out_type=jax.ShapeDtypeStruct((E, CW), jnp.float32),
        scratch_types=[
            pltpu.VMEM((GQ, CW), jnp.float32),
            pltpu.SemaphoreType.DMA,
        ],
    )
    return fn(contrib)


def _sc_scatter(contrib, idxp):
    mesh = plsc.VectorSubcoreMesh(core_axis_name="c", subcore_axis_name="s")
    fn = pl.kernel(
        _scatter_body,
        mesh=mesh,
        out_type=jax.ShapeDtypeStruct((OUT_ROWS, CW), jnp.float32),
        scratch_types=[
            pltpu.VMEM((GQ, CW), jnp.float32),
            pltpu.VMEM((1, GQ), jnp.int32),
            pltpu.VMEM_SHARED((ACC_ROWS, CW), jnp.float32),
            pltpu.SemaphoreType.DMA,
            pltpu.SemaphoreType.DMA,
        ],
    )
    zeros = jnp.zeros((ZCHUNK, CW), jnp.float32)
    return fn(contrib, idxp, zeros)


def _sc_zerodump_body(zeros_hbm, out_hbm, acc):
    c = lax.axis_index("c")
    s = lax.axis_index("s")
    for p_local in range(2):
        pbase = (2 * c + p_local) * PROWS
        pltpu.sync_copy(zeros_hbm, acc.at[pl.ds(s * ZCHUNK, ZCHUNK)])
        plsc.subcore_barrier()
        pltpu.sync_copy(
            acc.at[pl.ds(s * DCHUNK, DCHUNK)],
            out_hbm.at[pl.ds(pbase + s * DCHUNK, DCHUNK)])
        plsc.subcore_barrier()


def _sc_zerodump():
    mesh = plsc.VectorSubcoreMesh(core_axis_name="c", subcore_axis_name="s")
    fn = pl.kernel(
        _sc_zerodump_body,
        mesh=mesh,
        out_type=jax.ShapeDtypeStruct((OUT_ROWS, CW), jnp.float32),
        scratch_types=[
            pltpu.VMEM_SHARED((ACC_ROWS, CW), jnp.float32),
        ],
    )
    zeros = jnp.zeros((ZCHUNK, CW), jnp.float32)
    return fn(zeros)


def kernel(x, last_update, edge_index, t, msg,
           Wq, bq, Wk, bk, Wv, bv, We, Wskip, bskip, Wt, bt):
    edge_index = edge_index.astype(jnp.int32)
    src = edge_index[0]
    dst = edge_index[1]
    q, k, v, skip = _projections(x, Wq, bq, Wk, bk, Wv, bv, Wskip, bskip)
    ks, vs, qd, lus = _sc_gather(k, v, q, last_update, src, dst)
    wt_p = jnp.pad(Wt.reshape(1, TDIM), ((0, 0), (0, W - TDIM)))
    bt_p = jnp.pad(bt.reshape(1, TDIM), ((0, 0), (0, W - TDIM)))
    we1_p = jnp.pad(We[:TDIM], ((0, W - TDIM), (0, W - D)))
    we2_p = jnp.pad(We[TDIM:], ((0, 0), (0, W - D)))
    contrib = _edge_pass(lus.reshape(E, 1), t.reshape(E, 1), msg,
                         qd, ks, vs, wt_p, bt_p, we1_p, we2_p)
    acc = jax.ops.segment_sum(contrib, dst, num_segments=N)
    num = acc[:, :D]
    den = acc[:, D + 1:D + 3]
    denb = jnp.concatenate([
        jnp.repeat(den[:, 0:1], C, axis=1),
        jnp.repeat(den[:, 1:2], C, axis=1)], axis=1)
    out = num / (denb + 1e-16) + skip
    return out


# speedup vs baseline: 1.1843x; 1.1843x over previous
import functools

import jax
import jax.numpy as jnp
from jax import lax
from jax.experimental import pallas as pl
from jax.experimental.pallas import tpu as pltpu
from jax.experimental.pallas import tpu_sc as plsc

N = 50000
E = 800000
D = 100
H = 2
C = 50
TDIM = 100
MSG = 100
W = 128
CW = 128


PB = 2000


def _proj_kernel(x_ref, wq_ref, bq_ref, wk_ref, bk_ref, wv_ref, bv_ref,
                 ws_ref, bs_ref, q_ref, k_ref, v_ref, skip_ref):
    xb = x_ref[:]
    scale = 1.0 / (C ** 0.5)
    q_ref[:] = (jnp.dot(xb, wq_ref[:], preferred_element_type=jnp.float32)
                + bq_ref[:]) * scale
    k_ref[:] = jnp.dot(xb, wk_ref[:], preferred_element_type=jnp.float32) + bk_ref[:]
    v_ref[:] = jnp.dot(xb, wv_ref[:], preferred_element_type=jnp.float32) + bv_ref[:]
    skip_ref[:] = jnp.dot(xb, ws_ref[:], preferred_element_type=jnp.float32) + bs_ref[:]


def _projections(x, Wq, bq, Wk, bk, Wv, bv, Wskip, bskip):
    const = lambda i: (0, 0)
    blk = lambda i: (i, 0)
    w_spec = pl.BlockSpec((D, W), const)
    b_spec = pl.BlockSpec((1, W), const)
    o_spec = pl.BlockSpec((PB, W), blk)
    pad_w = lambda w: jnp.pad(w, ((0, 0), (0, W - D)))
    pad_b = lambda b: jnp.pad(b.reshape(1, D), ((0, 0), (0, W - D)))
    return pl.pallas_call(
        _proj_kernel,
        grid=(N // PB,),
        in_specs=[pl.BlockSpec((PB, D), blk),
                  w_spec, b_spec, w_spec, b_spec, w_spec, b_spec,
                  pl.BlockSpec((D, D), const), pl.BlockSpec((1, D), const)],
        out_specs=[o_spec, o_spec, o_spec, pl.BlockSpec((PB, D), blk)],
        out_shape=[jax.ShapeDtypeStruct((N, W), jnp.float32),
                   jax.ShapeDtypeStruct((N, W), jnp.float32),
                   jax.ShapeDtypeStruct((N, W), jnp.float32),
                   jax.ShapeDtypeStruct((N, D), jnp.float32)],
        compiler_params=pltpu.CompilerParams(
            dimension_semantics=("arbitrary",),
        ),
    )(x, pad_w(Wq), pad_b(bq), pad_w(Wk), pad_b(bk),
      pad_w(Wv), pad_b(bv), Wskip, bskip.reshape(1, D))


NW = 32
GQ = 128
NG = E // GQ
GPW = (NG + NW - 1) // NW


def _gather_body(k_hbm, v_hbm, q_hbm, lu_hbm, src_hbm, dst_hbm,
                 ks_hbm, vs_hbm, qd_hbm, lus_hbm,
                 sidx_v, didx_v, k_v, v_v, q_v, lu_v,
                 sem1, sem2, sem3, sem4):
    c = lax.axis_index("c")
    s = lax.axis_index("s")
    wid = s * 2 + c

    def body(i, _):
        g = wid + NW * i

        @pl.when(g < NG)
        def _():
            base = g * GQ
            pltpu.sync_copy(src_hbm.at[pl.ds(base, GQ)], sidx_v)
            pltpu.sync_copy(dst_hbm.at[pl.ds(base, GQ)], didx_v)
            cp1 = pltpu.async_copy(k_hbm.at[sidx_v], k_v, sem1)
            cp2 = pltpu.async_copy(v_hbm.at[sidx_v], v_v, sem2)
            cp3 = pltpu.async_copy(q_hbm.at[didx_v], q_v, sem3)
            cp4 = pltpu.async_copy(lu_hbm.at[sidx_v], lu_v, sem4)
            cp1.wait()
            cp2.wait()
            cp3.wait()
            cp4.wait()
            pltpu.sync_copy(k_v, ks_hbm.at[pl.ds(base, GQ)])
            pltpu.sync_copy(v_v, vs_hbm.at[pl.ds(base, GQ)])
            pltpu.sync_copy(q_v, qd_hbm.at[pl.ds(base, GQ)])
            pltpu.sync_copy(lu_v, lus_hbm.at[pl.ds(base, GQ)])
        return 0

    lax.fori_loop(0, GPW, body, 0)


def _sc_gather(k, v, q, lu, src, dst):
    mesh = plsc.VectorSubcoreMesh(core_axis_name="c", subcore_axis_name="s")
    fn = pl.kernel(
        _gather_body,
        mesh=mesh,
        out_type=[
            jax.ShapeDtypeStruct((E, W), jnp.float32),
            jax.ShapeDtypeStruct((E, W), jnp.float32),
            jax.ShapeDtypeStruct((E, W), jnp.float32),
            jax.ShapeDtypeStruct((E,), jnp.float32),
        ],
        scratch_types=[
            pltpu.VMEM((GQ,), jnp.int32),
            pltpu.VMEM((GQ,), jnp.int32),
            pltpu.VMEM((GQ, W), jnp.float32),
            pltpu.VMEM((GQ, W), jnp.float32),
            pltpu.VMEM((GQ, W), jnp.float32),
            pltpu.VMEM((GQ,), jnp.float32),
            pltpu.SemaphoreType.DMA,
            pltpu.SemaphoreType.DMA,
            pltpu.SemaphoreType.DMA,
            pltpu.SemaphoreType.DMA,
        ],
    )
    return fn(k, v, q, lu, src, dst)


EB = 2000
NB = E // EB


def _edge_pass_kernel(lus_ref, t_ref, msg_ref, qd_ref, ks_ref, vs_ref,
                      wt_ref, bt_ref, we1_ref, we2_ref, contrib_ref):
    rel = lus_ref[:] - t_ref[:]
    enc = jnp.cos(rel * wt_ref[:] + bt_ref[:])
    e = (jnp.dot(enc, we1_ref[:], preferred_element_type=jnp.float32)
         + jnp.dot(msg_ref[:], we2_ref[:], preferred_element_type=jnp.float32))
    kj = ks_ref[:] + e
    vj = vs_ref[:] + e
    prod = qd_ref[:] * kj
    lane = lax.broadcasted_iota(jnp.int32, (EB, W), 1)
    m0 = lane < C
    a0 = jnp.sum(jnp.where(m0, prod, 0.0), axis=1, keepdims=True)
    a1 = jnp.sum(jnp.where(m0, 0.0, prod), axis=1, keepdims=True)
    ex0 = jnp.exp(a0)
    ex1 = jnp.exp(a1)
    attnw = jnp.where(m0, ex0, ex1)
    base = attnw * vj
    base = jnp.where(lane == D + 1, ex0, base)
    base = jnp.where(lane == D + 2, ex1, base)
    contrib_ref[:] = base[:, :CW]


def _edge_pass(lus, t, msg, qd, ks, vs, wt, bt, we1, we2):
    const = lambda i: (0, 0)
    blk = lambda i: (i, 0)
    return pl.pallas_call(
        _edge_pass_kernel,
        grid=(NB,),
        in_specs=[
            pl.BlockSpec((EB, 1), blk),
            pl.BlockSpec((EB, 1), blk),
            pl.BlockSpec((EB, MSG), blk),
            pl.BlockSpec((EB, W), blk),
            pl.BlockSpec((EB, W), blk),
            pl.BlockSpec((EB, W), blk),
            pl.BlockSpec((1, W), const),
            pl.BlockSpec((1, W), const),
            pl.BlockSpec((W, W), const),
            pl.BlockSpec((MSG, W), const),
        ],
        out_specs=pl.BlockSpec((EB, CW), blk),
        out_shape=jax.ShapeDtypeStruct((E, CW), jnp.float32),
        compiler_params=pltpu.CompilerParams(
            dimension_semantics=("arbitrary",),
        ),
    )(lus, t, msg, qd, ks, vs, wt, bt, we1, we2)


PROWS = 12544
ACC_ROWS = PROWS
ZCHUNK = ACC_ROWS // 16
DCHUNK = 784
OUT_ROWS = 4 * PROWS


def _scatter_body(contrib_hbm, idxp_hbm, zeros_hbm, out_hbm,
                  rows0, idx_v, acc, semr0, semi0):
    c = lax.axis_index("c")
    s = lax.axis_index("s")

    SNI = (NG + 15) // 16

    for p_local in range(2):
        pp = 2 * c + p_local
        pbase = pp * PROWS
        pltpu.sync_copy(zeros_hbm, acc.at[pl.ds(s * ZCHUNK, ZCHUNK)])
        plsc.subcore_barrier()

        def body(i, _):
            g = s + 16 * i

            @pl.when(g < NG)
            def _():
                base = g * GQ
                row = pp * NG + g
                cpi = pltpu.async_copy(idxp_hbm.at[pl.ds(row, 1)],
                                       idx_v, semi0)
                cpr = pltpu.async_copy(contrib_hbm.at[pl.ds(base, GQ)], rows0,
                                       semr0)
                cpi.wait()
                cpr.wait()
                pltpu.sync_copy(rows0,
                                acc.at[plsc.Indices(idx_v.at[0],
                                                    ignored_value=-1)],
                                add=True)
            return 0

        lax.fori_loop(0, SNI, body, 0)
        plsc.subcore_barrier()
        pltpu.sync_copy(
            acc.at[pl.ds(s * DCHUNK, DCHUNK)],
            out_hbm.at[pl.ds(pbase + s * DCHUNK, DCHUNK)])
        plsc.subcore_barrier()


def _sc_copy_body(contrib_hbm, out_hbm, rows0, semr0):
    c = lax.axis_index("c")
    s = lax.axis_index("s")
    wid = s * 2 + c

    def body(i, _):
        g = wid + NW * i

        @pl.when(g < NG)
        def _():
            base = g * GQ
            pltpu.async_copy(contrib_hbm.at[pl.ds(base, GQ)], rows0,
                             semr0).wait()
            pltpu.sync_copy(rows0, out_hbm.at[pl.ds(base, GQ)])
        return 0

    lax.fori_loop(0, GPW, body, 0)


def _sc_copy(contrib):
    mesh = plsc.VectorSubcoreMesh(core_axis_name="c", subcore_axis_name="s")
    fn = pl.kernel(
        _sc_copy_body,
        mesh=mesh,
        out_type=jax.ShapeDtypeStruct((E, CW), jnp.float32),
        scratch_types=[
            pltpu.VMEM((GQ, CW), jnp.float32),
            pltpu.SemaphoreType.DMA,
        ],
    )
    return fn(contrib)


def _sc_scatter(contrib, idxp):
    mesh = plsc.VectorSubcoreMesh(core_axis_name="c", subcore_axis_name="s")
    fn = pl.kernel(
        _scatter_body,
        mesh=mesh,
        out_type=jax.ShapeDtypeStruct((OUT_ROWS, CW), jnp.float32),
        scratch_types=[
            pltpu.VMEM((GQ, CW), jnp.float32),
            pltpu.VMEM((1, GQ), jnp.int32),
            pltpu.VMEM_SHARED((ACC_ROWS, CW), jnp.float32),
            pltpu.SemaphoreType.DMA,
            pltpu.SemaphoreType.DMA,
        ],
    )
    zeros = jnp.zeros((ZCHUNK, CW), jnp.float32)
    return fn(contrib, idxp, zeros)


def _sc_zerodump_body(zeros_hbm, out_hbm, acc):
    c = lax.axis_index("c")
    s = lax.axis_index("s")
    for p_local in range(2):
        pbase = (2 * c + p_local) * PROWS
        pltpu.sync_copy(zeros_hbm, acc.at[pl.ds(s * ZCHUNK, ZCHUNK)])
        plsc.subcore_barrier()
        pltpu.sync_copy(
            acc.at[pl.ds(s * DCHUNK, DCHUNK)],
            out_hbm.at[pl.ds(pbase + s * DCHUNK, DCHUNK)])
        plsc.subcore_barrier()


def _sc_zerodump():
    mesh = plsc.VectorSubcoreMesh(core_axis_name="c", subcore_axis_name="s")
    fn = pl.kernel(
        _sc_zerodump_body,
        mesh=mesh,
        out_type=jax.ShapeDtypeStruct((OUT_ROWS, CW), jnp.float32),
        scratch_types=[
            pltpu.VMEM_SHARED((ACC_ROWS, CW), jnp.float32),
        ],
    )
    zeros = jnp.zeros((ZCHUNK, CW), jnp.float32)
    return fn(zeros)


def kernel(x, last_update, edge_index, t, msg,
           Wq, bq, Wk, bk, Wv, bv, We, Wskip, bskip, Wt, bt):
    edge_index = edge_index.astype(jnp.int32)
    src = edge_index[0]
    dst = edge_index[1]
    q, k, v, skip = _projections(x, Wq, bq, Wk, bk, Wv, bv, Wskip, bskip)
    ks, vs, qd, lus = _sc_gather(k, v, q, last_update, src, dst)
    wt_p = jnp.pad(Wt.reshape(1, TDIM), ((0, 0), (0, W - TDIM)))
    bt_p = jnp.pad(bt.reshape(1, TDIM), ((0, 0), (0, W - TDIM)))
    we1_p = jnp.pad(We[:TDIM], ((0, W - TDIM), (0, W - D)))
    we2_p = jnp.pad(We[TDIM:], ((0, 0), (0, W - D)))
    contrib = _edge_pass(lus.reshape(E, 1), t.reshape(E, 1), msg,
                         qd, ks, vs, wt_p, bt_p, we1_p, we2_p)
    pid = dst // PROWS
    rel = dst - pid * PROWS
    idxp = jnp.where(pid[None, :] == jnp.arange(4, dtype=jnp.int32)[:, None],
                     rel[None, :], -1).reshape(4 * NG, GQ)
    acc = _sc_scatter(contrib, idxp)[:N]
    num = acc[:, :D]
    den = acc[:, D + 1:D + 3]
    denb = jnp.concatenate([
        jnp.repeat(den[:, 0:1], C, axis=1),
        jnp.repeat(den[:, 1:2], C, axis=1)], axis=1)
    out = num / (denb + 1e-16) + skip
    return out


# speedup vs baseline: 1.2161x; 1.0268x over previous
import functools

import jax
import jax.numpy as jnp
from jax import lax
from jax.experimental import pallas as pl
from jax.experimental.pallas import tpu as pltpu
from jax.experimental.pallas import tpu_sc as plsc

N = 50000
E = 800000
D = 100
H = 2
C = 50
TDIM = 100
MSG = 100
W = 128
CW = 128


PB = 2000


def _proj_kernel(x_ref, lu_ref, wq_ref, bq_ref, wk_ref, bk_ref, wv_ref, bv_ref,
                 ws_ref, bs_ref, q_ref, k_ref, v_ref, skip_ref):
    xb = x_ref[:]
    scale = 1.0 / (C ** 0.5)
    q_ref[:] = (jnp.dot(xb, wq_ref[:], preferred_element_type=jnp.float32)
                + bq_ref[:]) * scale
    k_ref[:] = jnp.dot(xb, wk_ref[:], preferred_element_type=jnp.float32) + bk_ref[:]
    vd = jnp.dot(xb, wv_ref[:], preferred_element_type=jnp.float32) + bv_ref[:]
    lane = lax.broadcasted_iota(jnp.int32, (PB, W), 1)
    v_ref[:] = jnp.where(lane == D, lu_ref[:], vd)
    skip_ref[:] = jnp.dot(xb, ws_ref[:], preferred_element_type=jnp.float32) + bs_ref[:]


def _projections(x, lu, Wq, bq, Wk, bk, Wv, bv, Wskip, bskip):
    const = lambda i: (0, 0)
    blk = lambda i: (i, 0)
    w_spec = pl.BlockSpec((D, W), const)
    b_spec = pl.BlockSpec((1, W), const)
    o_spec = pl.BlockSpec((PB, W), blk)
    pad_w = lambda w: jnp.pad(w, ((0, 0), (0, W - D)))
    pad_b = lambda b: jnp.pad(b.reshape(1, D), ((0, 0), (0, W - D)))
    return pl.pallas_call(
        _proj_kernel,
        grid=(N // PB,),
        in_specs=[pl.BlockSpec((PB, D), blk), pl.BlockSpec((PB, 1), blk),
                  w_spec, b_spec, w_spec, b_spec, w_spec, b_spec,
                  pl.BlockSpec((D, D), const), pl.BlockSpec((1, D), const)],
        out_specs=[o_spec, o_spec, o_spec, pl.BlockSpec((PB, D), blk)],
        out_shape=[jax.ShapeDtypeStruct((N, W), jnp.float32),
                   jax.ShapeDtypeStruct((N, W), jnp.float32),
                   jax.ShapeDtypeStruct((N, W), jnp.float32),
                   jax.ShapeDtypeStruct((N, D), jnp.float32)],
        compiler_params=pltpu.CompilerParams(
            dimension_semantics=("arbitrary",),
        ),
    )(x, lu.reshape(N, 1), pad_w(Wq), pad_b(bq), pad_w(Wk), pad_b(bk),
      pad_w(Wv), pad_b(bv), Wskip, bskip.reshape(1, D))


NW = 32
GQ = 128
NG = E // GQ
GPW = (NG + NW - 1) // NW


def _gather_body(k_hbm, v_hbm, q_hbm, src_hbm, dst_hbm,
                 ks_hbm, vs_hbm, qd_hbm,
                 sidx_v, didx_v, k_v, v_v, q_v,
                 sem1, sem2, sem3):
    c = lax.axis_index("c")
    s = lax.axis_index("s")
    wid = s * 2 + c

    def body(i, _):
        g = wid + NW * i

        @pl.when(g < NG)
        def _():
            base = g * GQ
            pltpu.sync_copy(src_hbm.at[pl.ds(base, GQ)], sidx_v)
            pltpu.sync_copy(dst_hbm.at[pl.ds(base, GQ)], didx_v)
            cp1 = pltpu.async_copy(k_hbm.at[sidx_v], k_v, sem1)
            cp2 = pltpu.async_copy(v_hbm.at[sidx_v], v_v, sem2)
            cp3 = pltpu.async_copy(q_hbm.at[didx_v], q_v, sem3)
            cp1.wait()
            cp2.wait()
            cp3.wait()
            pltpu.sync_copy(k_v, ks_hbm.at[pl.ds(base, GQ)])
            pltpu.sync_copy(v_v, vs_hbm.at[pl.ds(base, GQ)])
            pltpu.sync_copy(q_v, qd_hbm.at[pl.ds(base, GQ)])
        return 0

    lax.fori_loop(0, GPW, body, 0)


def _sc_gather(k, v, q, src, dst):
    mesh = plsc.VectorSubcoreMesh(core_axis_name="c", subcore_axis_name="s")
    fn = pl.kernel(
        _gather_body,
        mesh=mesh,
        out_type=[
            jax.ShapeDtypeStruct((E, W), jnp.float32),
            jax.ShapeDtypeStruct((E, W), jnp.float32),
            jax.ShapeDtypeStruct((E, W), jnp.float32),
        ],
        scratch_types=[
            pltpu.VMEM((GQ,), jnp.int32),
            pltpu.VMEM((GQ,), jnp.int32),
            pltpu.VMEM((GQ, W), jnp.float32),
            pltpu.VMEM((GQ, W), jnp.float32),
            pltpu.VMEM((GQ, W), jnp.float32),
            pltpu.SemaphoreType.DMA,
            pltpu.SemaphoreType.DMA,
            pltpu.SemaphoreType.DMA,
        ],
    )
    return fn(k, v, q, src, dst)


EB = 2000
NB = E // EB


def _edge_pass_kernel(t_ref, msg_ref, qd_ref, ks_ref, vs_ref,
                      wt_ref, bt_ref, we1_ref, we2_ref, contrib_ref):
    rel = vs_ref[:, D:D + 1] - t_ref[:]
    enc = jnp.cos(rel * wt_ref[:] + bt_ref[:])
    e = (jnp.dot(enc, we1_ref[:], preferred_element_type=jnp.float32)
         + jnp.dot(msg_ref[:], we2_ref[:], preferred_element_type=jnp.float32))
    kj = ks_ref[:] + e
    vj = vs_ref[:] + e
    prod = qd_ref[:] * kj
    lane = lax.broadcasted_iota(jnp.int32, (EB, W), 1)
    m0 = lane < C
    a0 = jnp.sum(jnp.where(m0, prod, 0.0), axis=1, keepdims=True)
    a1 = jnp.sum(jnp.where(m0, 0.0, prod), axis=1, keepdims=True)
    ex0 = jnp.exp(a0)
    ex1 = jnp.exp(a1)
    attnw = jnp.where(m0, ex0, ex1)
    base = attnw * vj
    base = jnp.where(lane == D + 1, ex0, base)
    base = jnp.where(lane == D + 2, ex1, base)
    contrib_ref[:] = base[:, :CW]


def _edge_pass(t, msg, qd, ks, vs, wt, bt, we1, we2):
    const = lambda i: (0, 0)
    blk = lambda i: (i, 0)
    return pl.pallas_call(
        _edge_pass_kernel,
        grid=(NB,),
        in_specs=[
            pl.BlockSpec((EB, 1), blk),
            pl.BlockSpec((EB, MSG), blk),
            pl.BlockSpec((EB, W), blk),
            pl.BlockSpec((EB, W), blk),
            pl.BlockSpec((EB, W), blk),
            pl.BlockSpec((1, W), const),
            pl.BlockSpec((1, W), const),
            pl.BlockSpec((W, W), const),
            pl.BlockSpec((MSG, W), const),
        ],
        out_specs=pl.BlockSpec((EB, CW), blk),
        out_shape=jax.ShapeDtypeStruct((E, CW), jnp.float32),
        compiler_params=pltpu.CompilerParams(
            dimension_semantics=("arbitrary",),
        ),
    )(t, msg, qd, ks, vs, wt, bt, we1, we2)


PROWS = 12544
ACC_ROWS = PROWS
ZCHUNK = ACC_ROWS // 16
DCHUNK = 784
OUT_ROWS = 4 * PROWS


def _scatter_body(contrib_hbm, idxp_hbm, zeros_hbm, out_hbm,
                  rows0, idx_v, acc, semr0, semi0):
    c = lax.axis_index("c")
    s = lax.axis_index("s")

    SNI = (NG + 15) // 16

    for p_local in range(2):
        pp = 2 * c + p_local
        pbase = pp * PROWS
        pltpu.sync_copy(zeros_hbm, acc.at[pl.ds(s * ZCHUNK, ZCHUNK)])
        plsc.subcore_barrier()

        def body(i, _):
            g = s + 16 * i

            @pl.when(g < NG)
            def _():
                base = g * GQ
                row = pp * NG + g
                cpi = pltpu.async_copy(idxp_hbm.at[pl.ds(row, 1)],
                                       idx_v, semi0)
                cpr = pltpu.async_copy(contrib_hbm.at[pl.ds(base, GQ)], rows0,
                                       semr0)
                cpi.wait()
                cpr.wait()
                pltpu.sync_copy(rows0,
                                acc.at[plsc.Indices(idx_v.at[0],
                                                    ignored_value=-1)],
                                add=True)
            return 0

        lax.fori_loop(0, SNI, body, 0)
        plsc.subcore_barrier()
        pltpu.sync_copy(
            acc.at[pl.ds(s * DCHUNK, DCHUNK)],
            out_hbm.at[pl.ds(pbase + s * DCHUNK, DCHUNK)])
        plsc.subcore_barrier()


def _sc_copy_body(contrib_hbm, out_hbm, rows0, semr0):
    c = lax.axis_index("c")
    s = lax.axis_index("s")
    wid = s * 2 + c

    def body(i, _):
        g = wid + NW * i

        @pl.when(g < NG)
        def _():
            base = g * GQ
            pltpu.async_copy(contrib_hbm.at[pl.ds(base, GQ)], rows0,
                             semr0).wait()
            pltpu.sync_copy(rows0, out_hbm.at[pl.ds(base, GQ)])
        return 0

    lax.fori_loop(0, GPW, body, 0)


def _sc_copy(contrib):
    mesh = plsc.VectorSubcoreMesh(core_axis_name="c", subcore_axis_name="s")
    fn = pl.kernel(
        _sc_copy_body,
        mesh=mesh,
        out_type=jax.ShapeDtypeStruct((E, CW), jnp.float32),
        scratch_types=[
            pltpu.VMEM((GQ, CW), jnp.float32),
            pltpu.SemaphoreType.DMA,
        ],
    )
    return fn(contrib)


def _sc_scatter(contrib, idxp):
    mesh = plsc.VectorSubcoreMesh(core_axis_name="c", subcore_axis_name="s")
    fn = pl.kernel(
        _scatter_body,
        mesh=mesh,
        out_type=jax.ShapeDtypeStruct((OUT_ROWS, CW), jnp.float32),
        scratch_types=[
            pltpu.VMEM((GQ, CW), jnp.float32),
            pltpu.VMEM((1, GQ), jnp.int32),
            pltpu.VMEM_SHARED((ACC_ROWS, CW), jnp.float32),
            pltpu.SemaphoreType.DMA,
            pltpu.SemaphoreType.DMA,
        ],
    )
    zeros = jnp.zeros((ZCHUNK, CW), jnp.float32)
    return fn(contrib, idxp, zeros)


def _sc_zerodump_body(zeros_hbm, out_hbm, acc):
    c = lax.axis_index("c")
    s = lax.axis_index("s")
    for p_local in range(2):
        pbase = (2 * c + p_local) * PROWS
        pltpu.sync_copy(zeros_hbm, acc.at[pl.ds(s * ZCHUNK, ZCHUNK)])
        plsc.subcore_barrier()
        pltpu.sync_copy(
            acc.at[pl.ds(s * DCHUNK, DCHUNK)],
            out_hbm.at[pl.ds(pbase + s * DCHUNK, DCHUNK)])
        plsc.subcore_barrier()


def _sc_zerodump():
    mesh = plsc.VectorSubcoreMesh(core_axis_name="c", subcore_axis_name="s")
    fn = pl.kernel(
        _sc_zerodump_body,
        mesh=mesh,
        out_type=jax.ShapeDtypeStruct((OUT_ROWS, CW), jnp.float32),
        scratch_types=[
            pltpu.VMEM_SHARED((ACC_ROWS, CW), jnp.float32),
        ],
    )
    zeros = jnp.zeros((ZCHUNK, CW), jnp.float32)
    return fn(zeros)


def kernel(x, last_update, edge_index, t, msg,
           Wq, bq, Wk, bk, Wv, bv, We, Wskip, bskip, Wt, bt):
    edge_index = edge_index.astype(jnp.int32)
    src = edge_index[0]
    dst = edge_index[1]
    q, k, v, skip = _projections(x, last_update, Wq, bq, Wk, bk, Wv, bv,
                                 Wskip, bskip)
    ks, vs, qd = _sc_gather(k, v, q, src, dst)
    wt_p = jnp.pad(Wt.reshape(1, TDIM), ((0, 0), (0, W - TDIM)))
    bt_p = jnp.pad(bt.reshape(1, TDIM), ((0, 0), (0, W - TDIM)))
    we1_p = jnp.pad(We[:TDIM], ((0, W - TDIM), (0, W - D)))
    we2_p = jnp.pad(We[TDIM:], ((0, 0), (0, W - D)))
    contrib = _edge_pass(t.reshape(E, 1), msg,
                         qd, ks, vs, wt_p, bt_p, we1_p, we2_p)
    pid = dst // PROWS
    rel = dst - pid * PROWS
    idxp = jnp.where(pid[None, :] == jnp.arange(4, dtype=jnp.int32)[:, None],
                     rel[None, :], -1).reshape(4 * NG, GQ)
    acc = _sc_scatter(contrib, idxp)[:N]
    num = acc[:, :D]
    den = acc[:, D + 1:D + 3]
    denb = jnp.concatenate([
        jnp.repeat(den[:, 0:1], C, axis=1),
        jnp.repeat(den[:, 1:2], C, axis=1)], axis=1)
    out = num / (denb + 1e-16) + skip
    return out
